# Initial kernel scaffold; baseline (speedup 1.0000x reference)
import jax
import jax.numpy as jnp
from jax.experimental import pallas as pl


def kernel(x, Wg, bg, W1, b1, g1, be1, W2, b2):
    raise NotImplementedError("write your pallas kernel here")


# speedup vs baseline: 3.2519x; 3.2519x over previous
import functools

import jax
import jax.numpy as jnp
from jax.experimental import pallas as pl

N = 2048
D = 768
H = 768
E = 8
BN = 256


def _moe_body(x_ref, wg_ref, bg_ref, w1_ref, b1_ref, g1_ref, be1_ref,
              w2_ref, b2_ref, out_ref):
    xb = x_ref[...]

    logits = jnp.dot(xb, wg_ref[...], preferred_element_type=jnp.float32)
    logits = logits + bg_ref[...]
    neg_inf = jnp.float32(-jnp.inf)
    iota = jax.lax.broadcasted_iota(jnp.int32, logits.shape, 1)
    m1 = jnp.max(logits, axis=-1, keepdims=True)
    eq1 = logits == m1
    i1 = jnp.min(jnp.where(eq1, iota, E), axis=-1, keepdims=True)
    first1 = iota == i1
    l2 = jnp.where(first1, neg_inf, logits)
    m2 = jnp.max(l2, axis=-1, keepdims=True)
    eq2 = l2 == m2
    i2 = jnp.min(jnp.where(eq2, iota, E), axis=-1, keepdims=True)
    first2 = iota == i2
    sel = first1 | first2
    wsel = jnp.where(sel, jnp.exp(logits - m1), 0.0)
    cw = wsel / jnp.sum(wsel, axis=-1, keepdims=True)

    xb16 = xb.astype(jnp.bfloat16)
    acc = jnp.zeros((xb.shape[0], H), dtype=jnp.float32)
    for e in range(E):
        h = jnp.dot(xb16, w1_ref[e], preferred_element_type=jnp.float32)
        h = h + b1_ref[e][None, :]
        mu = jnp.mean(h, axis=-1, keepdims=True)
        var = jnp.mean((h - mu) ** 2, axis=-1, keepdims=True)
        h = (h - mu) * jax.lax.rsqrt(var + 1e-5)
        h = h * g1_ref[e][None, :] + be1_ref[e][None, :]
        h = h * 0.5 * (1.0 + jax.lax.erf(h * jnp.float32(0.7071067811865476)))
        y = jnp.dot(h.astype(jnp.bfloat16), w2_ref[e],
                    preferred_element_type=jnp.float32)
        y = y + b2_ref[e][None, :]
        acc = acc + cw[:, e][:, None] * y
    out_ref[...] = acc


@jax.jit
def kernel(x, Wg, bg, W1, b1, g1, be1, W2, b2):
    w1b = W1.astype(jnp.bfloat16)
    w2b = W2.astype(jnp.bfloat16)
    grid = (N // BN,)
    const = lambda i: (0, 0)
    const3 = lambda i: (0, 0, 0)
    out = pl.pallas_call(
        _moe_body,
        grid=grid,
        in_specs=[
            pl.BlockSpec((BN, D), lambda i: (i, 0)),
            pl.BlockSpec((D, E), const),
            pl.BlockSpec((1, E), const),
            pl.BlockSpec((E, D, H), const3),
            pl.BlockSpec((E, H), const),
            pl.BlockSpec((E, H), const),
            pl.BlockSpec((E, H), const),
            pl.BlockSpec((E, D, H), const3),
            pl.BlockSpec((E, H), const),
        ],
        out_specs=pl.BlockSpec((BN, H), lambda i: (i, 0)),
        out_shape=jax.ShapeDtypeStruct((N, H), jnp.float32),
    )(x, Wg, bg.reshape(1, E), w1b, b1, g1, be1, w2b, b2)
    return out


# speedup vs baseline: 3.5840x; 1.1021x over previous
import functools

import jax
import jax.numpy as jnp
from jax.experimental import pallas as pl

N = 2048
D = 768
H = 768
E = 8
BN = 512


def _moe_body(x_ref, wg_ref, bg_ref, w1_ref, b1_ref, g1_ref, be1_ref,
              w2_ref, b2_ref, out_ref):
    xb = x_ref[...]

    logits = jnp.dot(xb, wg_ref[...], preferred_element_type=jnp.float32)
    logits = logits + bg_ref[...]
    neg_inf = jnp.float32(-jnp.inf)
    iota = jax.lax.broadcasted_iota(jnp.int32, logits.shape, 1)
    m1 = jnp.max(logits, axis=-1, keepdims=True)
    eq1 = logits == m1
    i1 = jnp.min(jnp.where(eq1, iota, E), axis=-1, keepdims=True)
    first1 = iota == i1
    l2 = jnp.where(first1, neg_inf, logits)
    m2 = jnp.max(l2, axis=-1, keepdims=True)
    eq2 = l2 == m2
    i2 = jnp.min(jnp.where(eq2, iota, E), axis=-1, keepdims=True)
    first2 = iota == i2
    sel = first1 | first2
    wsel = jnp.where(sel, jnp.exp(logits - m1), 0.0)
    cw = wsel / jnp.sum(wsel, axis=-1, keepdims=True)

    xb16 = xb.astype(jnp.bfloat16)
    acc = jnp.zeros((xb.shape[0], H), dtype=jnp.float32)
    for e in range(E):
        h = jnp.dot(xb16, w1_ref[e], preferred_element_type=jnp.float32)
        h = h + b1_ref[e][None, :]
        mu = jnp.mean(h, axis=-1, keepdims=True)
        var = jnp.mean((h - mu) ** 2, axis=-1, keepdims=True)
        h = (h - mu) * jax.lax.rsqrt(var + 1e-5)
        h = h * g1_ref[e][None, :] + be1_ref[e][None, :]
        h = h * 0.5 * (1.0 + jax.lax.erf(h * jnp.float32(0.7071067811865476)))
        y = jnp.dot(h.astype(jnp.bfloat16), w2_ref[e],
                    preferred_element_type=jnp.float32)
        y = y + b2_ref[e][None, :]
        acc = acc + cw[:, e][:, None] * y
    out_ref[...] = acc


@jax.jit
def kernel(x, Wg, bg, W1, b1, g1, be1, W2, b2):
    w1b = W1.astype(jnp.bfloat16)
    w2b = W2.astype(jnp.bfloat16)
    grid = (N // BN,)
    const = lambda i: (0, 0)
    const3 = lambda i: (0, 0, 0)
    out = pl.pallas_call(
        _moe_body,
        grid=grid,
        in_specs=[
            pl.BlockSpec((BN, D), lambda i: (i, 0)),
            pl.BlockSpec((D, E), const),
            pl.BlockSpec((1, E), const),
            pl.BlockSpec((E, D, H), const3),
            pl.BlockSpec((E, H), const),
            pl.BlockSpec((E, H), const),
            pl.BlockSpec((E, H), const),
            pl.BlockSpec((E, D, H), const3),
            pl.BlockSpec((E, H), const),
        ],
        out_specs=pl.BlockSpec((BN, H), lambda i: (i, 0)),
        out_shape=jax.ShapeDtypeStruct((N, H), jnp.float32),
    )(x, Wg, bg.reshape(1, E), w1b, b1, g1, be1, w2b, b2)
    return out


# speedup vs baseline: 3.6746x; 1.0253x over previous
import functools

import jax
import jax.numpy as jnp
from jax.experimental import pallas as pl

N = 2048
D = 768
H = 768
E = 8
BN = 1024


def _moe_body(x_ref, wg_ref, bg_ref, w1_ref, b1_ref, g1_ref, be1_ref,
              w2_ref, b2_ref, out_ref):
    xb = x_ref[...]

    logits = jnp.dot(xb, wg_ref[...], preferred_element_type=jnp.float32)
    logits = logits + bg_ref[...]
    neg_inf = jnp.float32(-jnp.inf)
    iota = jax.lax.broadcasted_iota(jnp.int32, logits.shape, 1)
    m1 = jnp.max(logits, axis=-1, keepdims=True)
    eq1 = logits == m1
    i1 = jnp.min(jnp.where(eq1, iota, E), axis=-1, keepdims=True)
    first1 = iota == i1
    l2 = jnp.where(first1, neg_inf, logits)
    m2 = jnp.max(l2, axis=-1, keepdims=True)
    eq2 = l2 == m2
    i2 = jnp.min(jnp.where(eq2, iota, E), axis=-1, keepdims=True)
    first2 = iota == i2
    sel = first1 | first2
    wsel = jnp.where(sel, jnp.exp(logits - m1), 0.0)
    cw = wsel / jnp.sum(wsel, axis=-1, keepdims=True)

    xb16 = xb.astype(jnp.bfloat16)
    acc = jnp.zeros((xb.shape[0], H), dtype=jnp.float32)
    for e in range(E):
        h = jnp.dot(xb16, w1_ref[e], preferred_element_type=jnp.float32)
        h = h + b1_ref[e][None, :]
        mu = jnp.mean(h, axis=-1, keepdims=True)
        var = jnp.mean((h - mu) ** 2, axis=-1, keepdims=True)
        h = (h - mu) * jax.lax.rsqrt(var + 1e-5)
        h = h * g1_ref[e][None, :] + be1_ref[e][None, :]
        h = h * 0.5 * (1.0 + jax.lax.erf(h * jnp.float32(0.7071067811865476)))
        y = jnp.dot(h.astype(jnp.bfloat16), w2_ref[e],
                    preferred_element_type=jnp.float32)
        y = y + b2_ref[e][None, :]
        acc = acc + cw[:, e][:, None] * y
    out_ref[...] = acc


@jax.jit
def kernel(x, Wg, bg, W1, b1, g1, be1, W2, b2):
    w1b = W1.astype(jnp.bfloat16)
    w2b = W2.astype(jnp.bfloat16)
    grid = (N // BN,)
    const = lambda i: (0, 0)
    const3 = lambda i: (0, 0, 0)
    out = pl.pallas_call(
        _moe_body,
        grid=grid,
        in_specs=[
            pl.BlockSpec((BN, D), lambda i: (i, 0)),
            pl.BlockSpec((D, E), const),
            pl.BlockSpec((1, E), const),
            pl.BlockSpec((E, D, H), const3),
            pl.BlockSpec((E, H), const),
            pl.BlockSpec((E, H), const),
            pl.BlockSpec((E, H), const),
            pl.BlockSpec((E, D, H), const3),
            pl.BlockSpec((E, H), const),
        ],
        out_specs=pl.BlockSpec((BN, H), lambda i: (i, 0)),
        out_shape=jax.ShapeDtypeStruct((N, H), jnp.float32),
    )(x, Wg, bg.reshape(1, E), w1b, b1, g1, be1, w2b, b2)
    return out
